# Initial kernel scaffold; baseline (speedup 1.0000x reference)
import jax
import jax.numpy as jnp
from jax.experimental import pallas as pl


def kernel(x, edge_index, e_type, e_feat, Wm1_0, bm1_0, Wm2_0, bm2_0, Ws_0, bs_0, g_0, b_0, Wm1_1, bm1_1, Wm2_1, bm2_1, Ws_1, bs_1, g_1, b_1, Wa, ba, Wc1, bc1, Wc2, bc2):
    raise NotImplementedError("write your pallas kernel here")


# speedup vs baseline: 1.0216x; 1.0216x over previous
import functools

import jax
import jax.numpy as jnp
from jax import lax
from jax.experimental import pallas as pl
from jax.experimental.pallas import tpu as pltpu
from jax.experimental.pallas import tpu_sc as plsc


def _split_quarters(ref, y, col0):
    qw = ref.shape[2]
    for q in range(ref.shape[0]):
        ref[q] = y[:, col0 + q * qw:col0 + (q + 1) * qw]


def _edge_weights_body(ef_ref, et_ref, wf_ref, sel0_ref, sel1_ref, bias_ref,
                       out0_ref, out1_ref):
    y = jnp.dot(ef_ref[...], wf_ref[...], preferred_element_type=jnp.float32)
    y = y + jnp.where(et_ref[...] == 0.0, sel0_ref[...], sel1_ref[...])
    y = y + bias_ref[...]
    h = out0_ref.shape[0] * out0_ref.shape[2]
    _split_quarters(out0_ref, y, 0)
    _split_quarters(out1_ref, y, h)


def _node_pre_body(x_ref, w_ref, u_ref, xs_ref):
    y = jnp.dot(x_ref[...], w_ref[...], preferred_element_type=jnp.float32)
    h = u_ref.shape[0] * u_ref.shape[2]
    _split_quarters(u_ref, y, 0)
    xs_ref[...] = y[:, h:]


def _layer_norm(o, g, b):
    mu = jnp.mean(o, axis=1, keepdims=True)
    d = o - mu
    var = jnp.mean(d * d, axis=1, keepdims=True)
    return d * lax.rsqrt(var + 1e-5) * g + b


def _mid_layer(xs_ref, s_ref, hist_ref, wm2_ref, bm2_ref, bs_ref, g_ref,
               b_ref):
    s_cat = jnp.concatenate([s_ref[q] for q in range(s_ref.shape[0])], axis=1)
    deg = hist_ref[...][:, 0:1]
    pre = (xs_ref[...] + bs_ref[...]
           + jnp.dot(s_cat, wm2_ref[...], preferred_element_type=jnp.float32)
           + deg * bm2_ref[...])
    return _layer_norm(jnp.maximum(pre, 0.0), g_ref[...], b_ref[...])


def _finish_mid_body(xs_ref, s_ref, hist_ref, wm2_ref, bm2_ref, bs_ref,
                     g_ref, b_ref, wnext_ref, u_ref, xs1_ref):
    h = _mid_layer(xs_ref, s_ref, hist_ref, wm2_ref, bm2_ref, bs_ref, g_ref,
                   b_ref)
    y = jnp.dot(h, wnext_ref[...], preferred_element_type=jnp.float32)
    hd = u_ref.shape[0] * u_ref.shape[2]
    _split_quarters(u_ref, y, 0)
    xs1_ref[...] = y[:, hd:]


def _finish_last_body(xs_ref, s_ref, hist_ref, wm2_ref, bm2_ref, bs_ref,
                      g_ref, b_ref, h2_ref):
    h2_ref[...] = _mid_layer(xs_ref, s_ref, hist_ref, wm2_ref, bm2_ref,
                             bs_ref, g_ref, b_ref)


def _pool_body(h2_ref, wa_ref, wc1_ref, bc1_ref, wc2_ref, bc2_ref, out_ref,
               m_ref, z_ref, r_ref):
    i = pl.program_id(0)
    nb = pl.num_programs(0)

    @pl.when(i == 0)
    def _():
        m_ref[0] = -1e30
        z_ref[0] = 0.0
        r_ref[...] = jnp.zeros_like(r_ref)

    h2 = h2_ref[...]
    sc = jnp.dot(h2, wa_ref[...], preferred_element_type=jnp.float32)[:, 0:1]
    bm = jnp.max(sc)
    m_old = m_ref[0]
    m_new = jnp.maximum(m_old, bm)
    scale = jnp.exp(m_old - m_new)
    w = jnp.exp(sc - m_new)
    z_ref[0] = z_ref[0] * scale + jnp.sum(w)
    r_ref[...] = r_ref[...] * scale + jnp.sum(h2 * w, axis=0, keepdims=True)
    m_ref[0] = m_new

    @pl.when(i == nb - 1)
    def _():
        readout = r_ref[...] / z_ref[0]
        t = jnp.maximum(
            jnp.dot(readout, wc1_ref[...], preferred_element_type=jnp.float32)
            + bc1_ref[...], 0.0)
        out_ref[...] = (jnp.dot(t, wc2_ref[...],
                                preferred_element_type=jnp.float32)
                        + bc2_ref[...])


def _make_sc_segment(n_nodes, n_pad, n_edges, qw):
    NC, NS, L = 2, 16, 16
    K = 80
    EPT = n_edges // NS
    RPT = n_pad // NS
    ZR = 128
    assert EPT % K == 0 and RPT % ZR == 0 and RPT % 8 == 0
    mesh = plsc.VectorSubcoreMesh(core_axis_name="c", subcore_axis_name="s")

    @functools.partial(
        pl.kernel,
        out_type=[jax.ShapeDtypeStruct((2 * NC * n_pad, qw), jnp.float32),
                  jax.ShapeDtypeStruct((n_pad, L), jnp.float32)],
        mesh=mesh,
        compiler_params=pltpu.CompilerParams(use_tc_tiling_on_sc=False),
        scratch_types=[
            pltpu.VMEM((1, K), jnp.int32),
            pltpu.VMEM((1, K), jnp.int32),
            pltpu.VMEM((K, qw), jnp.float32),
            pltpu.VMEM((K, qw), jnp.float32),
            pltpu.VMEM((K, L), jnp.float32),
            pltpu.VMEM((ZR, qw), jnp.float32),
            pltpu.VMEM((ZR, L), jnp.float32),
            pltpu.VMEM_SHARED((n_pad, qw), jnp.float32),
            pltpu.VMEM_SHARED((n_pad, L), jnp.float32),
        ],
    )
    def sc_segment(u_hbm, ew_hbm, src_hbm, dst_hbm, s_out, hist_out,
                   src_v, dst_v, ew_v, row_v, ones_v, zer_v, zh_v, s_sh, h_sh):
        c = lax.axis_index("c")
        s = lax.axis_index("s")
        row0 = s * RPT
        base0 = s * EPT

        @pl.loop(0, ZR)
        def _(i):
            for j in range(qw // L):
                zer_v[i, pl.ds(j * L, L)] = jnp.zeros((L,), jnp.float32)
            zh_v[i, pl.ds(0, L)] = jnp.zeros((L,), jnp.float32)

        @pl.loop(0, K)
        def _(i):
            ones_v[i, pl.ds(0, L)] = jnp.full((L,), 1.0, jnp.float32)

        for q in range(2):
            quarter = 2 * c + q
            for k in range(RPT // ZR):
                pltpu.sync_copy(zer_v, s_sh.at[pl.ds(row0 + k * ZR, ZR)])
            if q == 0:
                @pl.when(c == 0)
                def _():
                    for k in range(RPT // ZR):
                        pltpu.sync_copy(zh_v,
                                        h_sh.at[pl.ds(row0 + k * ZR, ZR)])
            plsc.subcore_barrier()

            @pl.loop(0, EPT, step=K)
            def _(off):
                b = base0 + off
                eb = quarter * n_edges + b
                pltpu.sync_copy(src_hbm.at[pl.ds(eb, K)], src_v.at[0])
                pltpu.sync_copy(dst_hbm.at[pl.ds(b, K)], dst_v.at[0])
                pltpu.sync_copy(ew_hbm.at[pl.ds(eb, K)], ew_v)
                pltpu.sync_copy(u_hbm.at[src_v.at[0]], row_v)

                @pl.loop(0, K)
                def _(i):
                    for j in range(qw // L):
                        sl = pl.ds(j * L, L)
                        row_v[i, sl] = jnp.maximum(
                            row_v[i, sl] + ew_v[i, sl], 0.0)

                pltpu.sync_copy(row_v, s_sh.at[dst_v.at[0]], add=True)

                if q == 0:
                    @pl.when(c == 0)
                    def _():
                        pltpu.sync_copy(ones_v, h_sh.at[dst_v.at[0]],
                                        add=True)

            plsc.subcore_barrier()

            qoff = quarter * n_pad
            for k in range(RPT // ZR):
                sl_sh = pl.ds(row0 + k * ZR, ZR)
                pltpu.sync_copy(s_sh.at[sl_sh],
                                s_out.at[pl.ds(qoff + row0 + k * ZR, ZR)])
            if q == 0:
                @pl.when(c == 0)
                def _():
                    for k in range(RPT // ZR):
                        sl_sh = pl.ds(row0 + k * ZR, ZR)
                        pltpu.sync_copy(h_sh.at[sl_sh], hist_out.at[sl_sh])

    return sc_segment


def kernel(x, edge_index, e_type, e_feat,
           Wm1_0, bm1_0, Wm2_0, bm2_0, Ws_0, bs_0, g_0, b_0,
           Wm1_1, bm1_1, Wm2_1, bm2_1, Ws_1, bs_1, g_1, b_1,
           Wa, ba, Wc1, bc1, Wc2, bc2):
    f32 = jnp.float32
    N, IN = x.shape
    E = edge_index.shape[1]
    ED = e_feat.shape[1]
    H = Wm2_0.shape[0]
    C = Wc2.shape[1]
    QW = H // 4
    BE = 4000
    BN = 2000
    NP = ((N + 2047) // 2048) * 2048

    src = edge_index[0]
    dst = edge_index[1]
    src4 = jnp.concatenate([src, src + N, src + 2 * N, src + 3 * N])
    et = e_type.astype(f32).reshape(E, 1)

    Wf = jnp.concatenate([Wm1_0[IN:IN + ED], Wm1_1[H:H + ED]], axis=1)
    sel0 = jnp.concatenate([Wm1_0[IN + ED], Wm1_1[H + ED]])[None, :]
    sel1 = jnp.concatenate([Wm1_0[IN + ED + 1], Wm1_1[H + ED + 1]])[None, :]
    biasc = jnp.concatenate([bm1_0, bm1_1])[None, :]
    Wcat0 = jnp.concatenate([Wm1_0[:IN], Ws_0], axis=1)
    Wcat1 = jnp.concatenate([Wm1_1[:H], Ws_1], axis=1)
    Wa_pad = jnp.pad(Wa, ((0, 0), (0, 128 - Wa.shape[1])))
    Wc2_pad = jnp.pad(Wc2, ((0, 0), (0, 128 - C)))
    bc2_pad = jnp.pad(bc2, ((0, 128 - C)))[None, :]

    full = lambda shape: pl.BlockSpec(shape, lambda i: tuple(0 for _ in shape))

    ew0, ew1 = pl.pallas_call(
        _edge_weights_body,
        grid=(E // BE,),
        in_specs=[
            pl.BlockSpec((BE, ED), lambda i: (i, 0)),
            pl.BlockSpec((BE, 1), lambda i: (i, 0)),
            full((ED, 2 * H)),
            full((1, 2 * H)),
            full((1, 2 * H)),
            full((1, 2 * H)),
        ],
        out_specs=[pl.BlockSpec((4, BE, QW), lambda i: (0, i, 0))] * 2,
        out_shape=[jax.ShapeDtypeStruct((4, E, QW), f32)] * 2,
    )(e_feat, et, Wf, sel0, sel1, biasc)

    u0, xs0 = pl.pallas_call(
        _node_pre_body,
        grid=(N // BN,),
        in_specs=[pl.BlockSpec((BN, IN), lambda i: (i, 0)), full((IN, 2 * H))],
        out_specs=[pl.BlockSpec((4, BN, QW), lambda i: (0, i, 0)),
                   pl.BlockSpec((BN, H), lambda i: (i, 0))],
        out_shape=[jax.ShapeDtypeStruct((4, N, QW), f32),
                   jax.ShapeDtypeStruct((N, H), f32)],
    )(x, Wcat0)

    sc_segment = _make_sc_segment(N, NP, E, QW)

    s0_flat, hist = sc_segment(u0.reshape(4 * N, QW), ew0.reshape(4 * E, QW),
                               src4, dst)

    u1, xs1 = pl.pallas_call(
        _finish_mid_body,
        grid=(N // BN,),
        in_specs=[
            pl.BlockSpec((BN, H), lambda i: (i, 0)),
            pl.BlockSpec((4, BN, QW), lambda i: (0, i, 0)),
            pl.BlockSpec((BN, 16), lambda i: (i, 0)),
            full((H, H)),
            full((1, H)),
            full((1, H)),
            full((1, H)),
            full((1, H)),
            full((H, 2 * H)),
        ],
        out_specs=[pl.BlockSpec((4, BN, QW), lambda i: (0, i, 0)),
                   pl.BlockSpec((BN, H), lambda i: (i, 0))],
        out_shape=[jax.ShapeDtypeStruct((4, N, QW), f32),
                   jax.ShapeDtypeStruct((N, H), f32)],
    )(xs0, s0_flat.reshape(4, NP, QW), hist, Wm2_0, bm2_0[None, :],
      bs_0[None, :], g_0[None, :], b_0[None, :], Wcat1)

    s1_flat, _hist1 = sc_segment(u1.reshape(4 * N, QW),
                                 ew1.reshape(4 * E, QW), src4, dst)

    h2 = pl.pallas_call(
        _finish_last_body,
        grid=(N // BN,),
        in_specs=[
            pl.BlockSpec((BN, H), lambda i: (i, 0)),
            pl.BlockSpec((4, BN, QW), lambda i: (0, i, 0)),
            pl.BlockSpec((BN, 16), lambda i: (i, 0)),
            full((H, H)),
            full((1, H)),
            full((1, H)),
            full((1, H)),
            full((1, H)),
        ],
        out_specs=pl.BlockSpec((BN, H), lambda i: (i, 0)),
        out_shape=jax.ShapeDtypeStruct((N, H), f32),
    )(xs1, s1_flat.reshape(4, NP, QW), hist, Wm2_1, bm2_1[None, :],
      bs_1[None, :], g_1[None, :], b_1[None, :])

    logits_pad = pl.pallas_call(
        _pool_body,
        grid=(N // BN,),
        in_specs=[
            pl.BlockSpec((BN, H), lambda i: (i, 0)),
            full((H, 128)),
            full((H, H)),
            full((1, H)),
            full((H, 128)),
            full((1, 128)),
        ],
        out_specs=pl.BlockSpec((1, 128), lambda i: (0, 0)),
        out_shape=jax.ShapeDtypeStruct((1, 128), f32),
        scratch_shapes=[
            pltpu.SMEM((1,), f32),
            pltpu.SMEM((1,), f32),
            pltpu.VMEM((1, H), f32),
        ],
    )(h2, Wa_pad, Wc1, bc1[None, :], Wc2_pad, bc2_pad)

    return logits_pad[0:1, 0:C]
